# baseline (speedup 1.0000x reference)
import functools

import jax
import jax.numpy as jnp
from jax import lax
from jax.experimental import pallas as pl
from jax.experimental.pallas import tpu as pltpu
from jax.experimental.pallas import tpu_sc as plsc

NC = 2
NS = 16
NW = NC * NS
WC = 512


def _i32(x):
  return jnp.asarray(x, jnp.int32)


def _floop(n, body):
  lax.fori_loop(_i32(0), _i32(n), lambda i, c: (body(i), c)[1], _i32(0),
                unroll=False)


LB = 16512
MW = 2 * LB + 128


def _win_owner(idv, wid):
  winidx = idv // WC
  return winidx, lax.rem(winidx, _i32(NW)) == wid


def _gather_body(ids2_hbm, memt_hbm, h_hbm, meta_hbm,
                 ids_v, tbl, l_b, l_id, wb, wl, win, wint, hst, cbuf, sem):
  rows = ids_v.shape[0]
  M = memt_hbm.shape[1]
  TBL = tbl.shape[0]
  nwin = (M + WC - 1) // WC
  wid = lax.axis_index("c") * NS + lax.axis_index("s")
  iota = lax.iota(jnp.int32, 16)
  zeros16 = jnp.zeros((16,), jnp.int32)

  def zero(i):
    tbl[pl.ds(i * 16, 16)] = zeros16
  _floop(TBL // 16, zero)

  pltpu.sync_copy(ids2_hbm, ids_v)

  def slot_of(idv):
    winidx, m = _win_owner(idv, wid)
    slot = (winidx // NW) * WC + lax.rem(idv, _i32(WC))
    return m, jnp.where(m, slot, 0)

  def p1(r):
    for o in range(8):
      b0 = r * 128 + o * 16
      idv = ids_v[r, pl.ds(o * 16, 16)]
      m, slot = slot_of(idv)
      plsc.store_scatter(tbl, [slot], iota + (b0 + 1), mask=m)
  _floop(rows, p1)

  def p2(r, cnt):
    for o in range(8):
      b0 = r * 128 + o * 16
      idv = ids_v[r, pl.ds(o * 16, 16)]
      m, slot = slot_of(idv)
      g = plsc.load_gather(tbl, [slot], mask=m)
      keep = m & (g == iota + (b0 + 1))
      plsc.store_compressed(l_b.at[pl.ds(cnt, 16)], iota + b0, mask=keep)
      plsc.store_compressed(l_id.at[pl.ds(cnt, 16)], idv, mask=keep)
      cnt = cnt + plsc.all_reduce_population_count(keep)[0]
    return cnt

  cnt = lax.fori_loop(_i32(0), _i32(rows), p2, _i32(0))
  nscan = (cnt + 15) // 16

  cbuf[pl.ds(0, 16)] = jnp.full((16,), cnt, jnp.int32)
  pltpu.sync_copy(l_b, meta_hbm.at[_i32(wid), pl.ds(0, LB)])
  pltpu.sync_copy(l_id, meta_hbm.at[_i32(wid), pl.ds(LB, LB)])
  pltpu.sync_copy(cbuf, meta_hbm.at[_i32(wid), pl.ds(2 * LB, 128)])

  def extract(widx, wref):
    c0 = widx * WC

    def scan(i, wcnt):
      off = i * 16
      valid = (off + iota) < cnt
      idv = l_id[pl.ds(off, 16)]
      inwin = valid & ((idv // WC) == widx)
      plsc.store_compressed(wb.at[pl.ds(wcnt, 16)], l_b[pl.ds(off, 16)],
                            mask=inwin)
      plsc.store_compressed(wl.at[pl.ds(wcnt, 16)],
                            jnp.where(inwin, idv - c0, 0), mask=inwin)
      return wcnt + plsc.all_reduce_population_count(inwin)[0]

    wcnt = lax.fori_loop(_i32(0), nscan, scan, _i32(0))

    def ext16(k):
      off = k * 16
      bv = wb[pl.ds(off, 16)]
      lv = wl[pl.ds(off, 16)]
      cps = []
      for j in range(16):
        ok = off + j < wcnt

        @pl.when(ok)
        def _pull(j=j):
          colv = jnp.full((16,), lv[j], jnp.int32)
          for q in range(4):
            hst[j, pl.ds(q * 16, 16)] = plsc.load_gather(
                wref, [iota + q * 16, colv])
        cp = pltpu.make_async_copy(
            hst.at[pl.ds(j, 1)], h_hbm.at[pl.ds(bv[j], 1)], sem)
        cps.append((ok, cp))
      for ok, cp in cps:
        @pl.when(ok)
        def _start(cp=cp):
          cp.start()
      for ok, cp in cps:
        @pl.when(ok)
        def _wait(cp=cp):
          cp.wait()

    _floop((wcnt + 15) // 16, ext16)

  def do_window(g):
    widx = wid + g * NW
    c0 = widx * WC
    is_tail = widx == nwin - 1

    @pl.when(jnp.logical_not(is_tail))
    def _full():
      pltpu.sync_copy(memt_hbm.at[:, pl.ds(c0, WC)], win)
      extract(widx, win)

    @pl.when(is_tail)
    def _tail():
      pltpu.sync_copy(memt_hbm.at[:, pl.ds(c0, 128)], wint)
      extract(widx, wint)

  gmax = (_i32(nwin - 1) - wid) // NW + 1
  _floop(gmax, do_window)


def _make_gather(M, D, B):
  rows = B // 128
  nwin = (M + WC - 1) // WC
  TBL = ((nwin + NW - 1) // NW) * WC
  mesh = plsc.VectorSubcoreMesh(core_axis_name="c", subcore_axis_name="s")
  return pl.kernel(
      _gather_body,
      out_type=(jax.ShapeDtypeStruct((B, D), jnp.float32),
                jax.ShapeDtypeStruct((NW, MW), jnp.int32)),
      mesh=mesh,
      compiler_params=pltpu.CompilerParams(needs_layout_passes=False,
                                           disable_bounds_checks=True),
      scratch_types=[
          pltpu.VMEM((rows, 128), jnp.int32),
          pltpu.VMEM((TBL,), jnp.int32),
          pltpu.VMEM((LB,), jnp.int32),
          pltpu.VMEM((LB,), jnp.int32),
          pltpu.VMEM((WC + 16,), jnp.int32),
          pltpu.VMEM((WC + 16,), jnp.int32),
          pltpu.VMEM((D, WC), jnp.float32),
          pltpu.VMEM((D, 128), jnp.float32),
          pltpu.VMEM((16, D), jnp.float32),
          pltpu.VMEM((128,), jnp.int32),
          pltpu.SemaphoreType.DMA,
      ],
  )


def _gru_body(x_ref, h_ref, wir, wiz, win, whr, whz, whn, br, bz, bni, bnh,
              out_ref):
  x = x_ref[...]
  h = h_ref[...]
  dot = functools.partial(
      lax.dot_general,
      dimension_numbers=(((1,), (0,)), ((), ())),
      preferred_element_type=jnp.float32)
  r = jax.nn.sigmoid(dot(x, wir[...]) + dot(h, whr[...]) + br[...])
  z = jax.nn.sigmoid(dot(x, wiz[...]) + dot(h, whz[...]) + bz[...])
  n = jnp.tanh(dot(x, win[...]) + bni[...] + r * (dot(h, whn[...]) + bnh[...]))
  res = (1.0 - z) * n + z * h
  out_ref[...] = jnp.concatenate([res, jnp.zeros_like(res)], axis=1)


def _make_gru(B, DM, D, blk):
  grid = (B // blk,)
  row_spec = lambda w: pl.BlockSpec((blk, w), lambda i: (i, _i32(0)))
  fix_spec = lambda a, b: pl.BlockSpec((a, b), lambda i: (_i32(0), _i32(0)))
  return pl.pallas_call(
      _gru_body,
      grid=grid,
      in_specs=[
          row_spec(DM),
          row_spec(D),
          fix_spec(DM, D), fix_spec(DM, D), fix_spec(DM, D),
          fix_spec(D, D), fix_spec(D, D), fix_spec(D, D),
          fix_spec(1, D), fix_spec(1, D), fix_spec(1, D), fix_spec(1, D),
      ],
      out_specs=row_spec(2 * D),
      out_shape=jax.ShapeDtypeStruct((B, 2 * D), jnp.float32),
  )


def _scatter_body(meta_hbm, ts_hbm, newh_hbm, memt_hbm, lu_ref,
                  outt_hbm,
                  ts_v, l_b, l_id, wb, wl, win, win2, wint, rows16, ts16,
                  cbuf, sem, insem):
  D = memt_hbm.shape[0]
  M = memt_hbm.shape[1]
  nwin = (M + WC - 1) // WC
  wid = lax.axis_index("c") * NS + lax.axis_index("s")
  iota = lax.iota(jnp.int32, 16)

  pltpu.sync_copy(ts_hbm, ts_v)
  pltpu.sync_copy(meta_hbm.at[_i32(wid), pl.ds(0, LB)], l_b)
  pltpu.sync_copy(meta_hbm.at[_i32(wid), pl.ds(LB, LB)], l_id)
  pltpu.sync_copy(meta_hbm.at[_i32(wid), pl.ds(2 * LB, 128)], cbuf)
  cnt = cbuf[pl.ds(0, 16)][0]
  nscan = (cnt + 15) // 16

  def lu16(k):
    off = k * 16
    valid = (off + iota) < cnt
    bv = l_b[pl.ds(off, 16)]
    idv = l_id[pl.ds(off, 16)]
    ts16[pl.ds(0, 16)] = plsc.load_gather(ts_v, [jnp.where(valid, bv, 0)])
    pltpu.async_copy(
        ts16, lu_ref.at[plsc.Indices(jnp.where(valid, idv, -1),
                                     ignored_value=-1)], sem).wait()
  _floop(nscan, lu16)

  def process(widx, wref):
    c0 = widx * WC

    def scan(i, wcnt):
      off = i * 16
      valid = (off + iota) < cnt
      idv = l_id[pl.ds(off, 16)]
      inwin = valid & ((idv // WC) == widx)
      plsc.store_compressed(wb.at[pl.ds(wcnt, 16)], l_b[pl.ds(off, 16)],
                            mask=inwin)
      plsc.store_compressed(wl.at[pl.ds(wcnt, 16)],
                            jnp.where(inwin, idv - c0, 0), mask=inwin)
      return wcnt + plsc.all_reduce_population_count(inwin)[0]

    wcnt = lax.fori_loop(_i32(0), nscan, scan, _i32(0))

    def apply16(k):
      off = k * 16
      valid = (off + iota) < wcnt
      bv = wb[pl.ds(off, 16)]
      local = wl[pl.ds(off, 16)]
      pltpu.async_copy(
          newh_hbm.at[plsc.Indices(jnp.where(valid, bv, -1),
                                   ignored_value=-1)],
          rows16, sem).wait()
      for j in range(16):
        @pl.when(off + j < wcnt)
        def _apply(j=j):
          colv = jnp.full((16,), local[j], jnp.int32)
          for q in range(4):
            plsc.store_scatter(wref, [iota + q * 16, colv],
                               rows16[j, pl.ds(q * 16, 16)])

    _floop((wcnt + 15) // 16, apply16)

  gmax = (_i32(nwin - 1) - wid) // NW + 1

  def in_cp(g, buf):
    widx = wid + g * NW
    c0 = widx * WC
    return pltpu.make_async_copy(memt_hbm.at[:, pl.ds(c0, WC)], buf, insem)

  def tail_cp(g):
    c0 = (wid + g * NW) * WC
    return pltpu.make_async_copy(memt_hbm.at[:, pl.ds(c0, 128)], wint, insem)

  def start_next(g, buf):
    @pl.when(g < gmax)
    def _():
      is_tail = wid + g * NW == nwin - 1

      @pl.when(jnp.logical_not(is_tail))
      def _f():
        in_cp(g, buf).start()

      @pl.when(is_tail)
      def _t():
        tail_cp(g).start()

  def run_window(g, buf):
    @pl.when(g < gmax)
    def _():
      widx = wid + g * NW
      c0 = widx * WC
      is_tail = widx == nwin - 1

      @pl.when(jnp.logical_not(is_tail))
      def _f():
        in_cp(g, buf).wait()
        process(widx, buf)
        pltpu.sync_copy(buf, outt_hbm.at[:, pl.ds(c0, WC)])

      @pl.when(is_tail)
      def _t():
        tail_cp(g).wait()
        process(widx, wint)
        pltpu.sync_copy(wint, outt_hbm.at[:, pl.ds(c0, 128)])

  start_next(_i32(0), win)

  def pair(t):
    g0 = t * 2
    start_next(g0 + 1, win2)
    run_window(g0, win)
    start_next(g0 + 2, win)
    run_window(g0 + 1, win2)

  _floop((gmax + 1) // 2, pair)


def _make_scatter(M, D, DM, B):
  mesh = plsc.VectorSubcoreMesh(core_axis_name="c", subcore_axis_name="s")
  return pl.kernel(
      _scatter_body,
      out_type=jax.ShapeDtypeStruct((D, M), jnp.float32),
      mesh=mesh,
      compiler_params=pltpu.CompilerParams(needs_layout_passes=False,
                                           disable_bounds_checks=True),
      scratch_types=[
          pltpu.VMEM((B,), jnp.float32),
          pltpu.VMEM((LB,), jnp.int32),
          pltpu.VMEM((LB,), jnp.int32),
          pltpu.VMEM((WC + 16,), jnp.int32),
          pltpu.VMEM((WC + 16,), jnp.int32),
          pltpu.VMEM((D, WC), jnp.float32),
          pltpu.VMEM((D, WC), jnp.float32),
          pltpu.VMEM((D, 128), jnp.float32),
          pltpu.VMEM((16, DM), jnp.float32),
          pltpu.VMEM((16,), jnp.float32),
          pltpu.VMEM((128,), jnp.int32),
          pltpu.SemaphoreType.DMA,
          pltpu.SemaphoreType.DMA,
      ],
  )


def kernel(unique_node_ids, unique_messages, timestamps, memory, last_update,
           W_ih, W_hh, b_ih, b_hh):
  M, D = memory.shape
  B, DM = unique_messages.shape

  ids2 = unique_node_ids.astype(jnp.int32).reshape(B // 128, 128)
  W_ihT = W_ih.T.astype(jnp.float32)
  W_hhT = W_hh.T.astype(jnp.float32)
  wir, wiz, win = W_ihT[:, :D], W_ihT[:, D:2 * D], W_ihT[:, 2 * D:]
  whr, whz, whn = W_hhT[:, :D], W_hhT[:, D:2 * D], W_hhT[:, 2 * D:]
  br = (b_ih[:D] + b_hh[:D]).reshape(1, D)
  bz = (b_ih[D:2 * D] + b_hh[D:2 * D]).reshape(1, D)
  bni = b_ih[2 * D:].reshape(1, D)
  bnh = b_hh[2 * D:].reshape(1, D)

  mem_t = memory.T

  h, meta = _make_gather(M, D, B)(ids2, mem_t)
  newh = _make_gru(B, DM, D, 512)(
      unique_messages, h, wir, wiz, win, whr, whz, whn, br, bz, bni, bnh)

  lu_ref = jax.new_ref(last_update)
  out_t = _make_scatter(M, D, DM, B)(meta, timestamps, newh, mem_t, lu_ref)
  return out_t.T, lu_ref[...]


# speedup vs baseline: 1.0476x; 1.0476x over previous
import functools

import jax
import jax.numpy as jnp
from jax import lax
from jax.experimental import pallas as pl
from jax.experimental.pallas import tpu as pltpu
from jax.experimental.pallas import tpu_sc as plsc

NC = 2
NS = 16
NW = NC * NS
WC = 512


def _i32(x):
  return jnp.asarray(x, jnp.int32)


def _floop(n, body):
  lax.fori_loop(_i32(0), _i32(n), lambda i, c: (body(i), c)[1], _i32(0),
                unroll=False)


LB = 16512
MW = 2 * LB + 128


def _win_owner(idv, wid):
  winidx = idv // WC
  return winidx, lax.rem(winidx, _i32(NW)) == wid


def _gather_body(ids2_hbm, memt_hbm, h_hbm, meta_hbm,
                 l_b, l_id, wb, wl, win, win2, wint, hst, cbuf, sem, insem):
  rows = ids2_hbm.shape[0]
  M = memt_hbm.shape[1]
  nwin = (M + WC - 1) // WC
  TBL = ((nwin + NW - 1) // NW) * WC
  wid = lax.axis_index("c") * NS + lax.axis_index("s")
  iota = lax.iota(jnp.int32, 16)
  zeros16 = jnp.zeros((16,), jnp.int32)

  def passes(ids_v, tbl):
    def zero(i):
      tbl[pl.ds(i * 16, 16)] = zeros16
    _floop(TBL // 16, zero)

    pltpu.sync_copy(ids2_hbm, ids_v)

    def slot_of(idv):
      winidx, m = _win_owner(idv, wid)
      slot = (winidx // NW) * WC + lax.rem(idv, _i32(WC))
      return m, jnp.where(m, slot, 0)

    def p1(r):
      for o in range(8):
        b0 = r * 128 + o * 16
        idv = ids_v[r, pl.ds(o * 16, 16)]
        m, slot = slot_of(idv)
        plsc.store_scatter(tbl, [slot], iota + (b0 + 1), mask=m)
    _floop(rows, p1)

    def p2(r, cnt):
      for o in range(8):
        b0 = r * 128 + o * 16
        idv = ids_v[r, pl.ds(o * 16, 16)]
        m, slot = slot_of(idv)
        g = plsc.load_gather(tbl, [slot], mask=m)
        keep = m & (g == iota + (b0 + 1))
        plsc.store_compressed(l_b.at[pl.ds(cnt, 16)], iota + b0, mask=keep)
        plsc.store_compressed(l_id.at[pl.ds(cnt, 16)], idv, mask=keep)
        cnt = cnt + plsc.all_reduce_population_count(keep)[0]
      return cnt

    return lax.fori_loop(_i32(0), _i32(rows), p2, _i32(0))

  cnt = pl.run_scoped(
      passes,
      pltpu.VMEM((rows, 128), jnp.int32),
      pltpu.VMEM((TBL,), jnp.int32))
  nscan = (cnt + 15) // 16

  cbuf[pl.ds(0, 16)] = jnp.full((16,), cnt, jnp.int32)
  pltpu.sync_copy(l_b, meta_hbm.at[_i32(wid), pl.ds(0, LB)])
  pltpu.sync_copy(l_id, meta_hbm.at[_i32(wid), pl.ds(LB, LB)])
  pltpu.sync_copy(cbuf, meta_hbm.at[_i32(wid), pl.ds(2 * LB, 128)])

  def extract(sub, wref):
    c0 = sub * (WC // 2)

    def scan(i, wcnt):
      off = i * 16
      valid = (off + iota) < cnt
      idv = l_id[pl.ds(off, 16)]
      inwin = valid & ((idv // (WC // 2)) == sub)
      plsc.store_compressed(wb.at[pl.ds(wcnt, 16)], l_b[pl.ds(off, 16)],
                            mask=inwin)
      plsc.store_compressed(wl.at[pl.ds(wcnt, 16)],
                            jnp.where(inwin, idv - c0, 0), mask=inwin)
      return wcnt + plsc.all_reduce_population_count(inwin)[0]

    wcnt = lax.fori_loop(_i32(0), nscan, scan, _i32(0))

    def ext16(k):
      off = k * 16
      bv = wb[pl.ds(off, 16)]
      lv = wl[pl.ds(off, 16)]
      cps = []
      for j in range(16):
        ok = off + j < wcnt

        @pl.when(ok)
        def _pull(j=j):
          colv = jnp.full((16,), lv[j], jnp.int32)
          for q in range(4):
            hst[j, pl.ds(q * 16, 16)] = plsc.load_gather(
                wref, [iota + q * 16, colv])
        cp = pltpu.make_async_copy(
            hst.at[pl.ds(j, 1)], h_hbm.at[pl.ds(bv[j], 1)], sem)
        cps.append((ok, cp))
      for ok, cp in cps:
        @pl.when(ok)
        def _start(cp=cp):
          cp.start()
      for ok, cp in cps:
        @pl.when(ok)
        def _wait(cp=cp):
          cp.wait()

    _floop((wcnt + 15) // 16, ext16)

  gmax = (_i32(nwin - 1) - wid) // NW + 1
  owns_tail = (wid == lax.rem(_i32(nwin - 1), _i32(NW))).astype(jnp.int32)
  hmax = 2 * gmax - owns_tail

  def sub_of(h):
    return 2 * wid + (h // 2) * (2 * NW) + lax.rem(h, _i32(2))

  def in_cp(h, buf):
    c0 = sub_of(h) * (WC // 2)
    return pltpu.make_async_copy(
        memt_hbm.at[:, pl.ds(c0, WC // 2)], buf, insem)

  def tail_cp(h):
    c0 = sub_of(h) * (WC // 2)
    return pltpu.make_async_copy(memt_hbm.at[:, pl.ds(c0, 128)], wint, insem)

  def is_tail_sub(h):
    return (owns_tail == 1) & (h == hmax - 1)

  def start_next(h, buf):
    @pl.when(h < hmax)
    def _():
      @pl.when(jnp.logical_not(is_tail_sub(h)))
      def _f():
        in_cp(h, buf).start()

      @pl.when(is_tail_sub(h))
      def _t():
        tail_cp(h).start()

  def run_window(h, buf):
    @pl.when(h < hmax)
    def _():
      @pl.when(jnp.logical_not(is_tail_sub(h)))
      def _f():
        in_cp(h, buf).wait()
        extract(sub_of(h), buf)

      @pl.when(is_tail_sub(h))
      def _t():
        tail_cp(h).wait()
        extract(sub_of(h), wint)

  start_next(_i32(0), win)

  def pair(t):
    h0 = t * 2
    start_next(h0 + 1, win2)
    run_window(h0, win)
    start_next(h0 + 2, win)
    run_window(h0 + 1, win2)

  _floop((hmax + 1) // 2, pair)


def _make_gather(M, D, B):
  rows = B // 128
  nwin = (M + WC - 1) // WC
  TBL = ((nwin + NW - 1) // NW) * WC
  mesh = plsc.VectorSubcoreMesh(core_axis_name="c", subcore_axis_name="s")
  return pl.kernel(
      _gather_body,
      out_type=(jax.ShapeDtypeStruct((B, D), jnp.float32),
                jax.ShapeDtypeStruct((NW, MW), jnp.int32)),
      mesh=mesh,
      compiler_params=pltpu.CompilerParams(needs_layout_passes=False,
                                           disable_bounds_checks=True),
      scratch_types=[
          pltpu.VMEM((LB,), jnp.int32),
          pltpu.VMEM((LB,), jnp.int32),
          pltpu.VMEM((WC + 16,), jnp.int32),
          pltpu.VMEM((WC + 16,), jnp.int32),
          pltpu.VMEM((D, WC // 2), jnp.float32),
          pltpu.VMEM((D, WC // 2), jnp.float32),
          pltpu.VMEM((D, 128), jnp.float32),
          pltpu.VMEM((16, D), jnp.float32),
          pltpu.VMEM((128,), jnp.int32),
          pltpu.SemaphoreType.DMA,
          pltpu.SemaphoreType.DMA,
      ],
  )


def _gru_body(x_ref, h_ref, wir, wiz, win, whr, whz, whn, br, bz, bni, bnh,
              out_ref):
  x = x_ref[...]
  h = h_ref[...]
  dot = functools.partial(
      lax.dot_general,
      dimension_numbers=(((1,), (0,)), ((), ())),
      preferred_element_type=jnp.float32)
  r = jax.nn.sigmoid(dot(x, wir[...]) + dot(h, whr[...]) + br[...])
  z = jax.nn.sigmoid(dot(x, wiz[...]) + dot(h, whz[...]) + bz[...])
  n = jnp.tanh(dot(x, win[...]) + bni[...] + r * (dot(h, whn[...]) + bnh[...]))
  res = (1.0 - z) * n + z * h
  out_ref[...] = jnp.concatenate([res, jnp.zeros_like(res)], axis=1)


def _make_gru(B, DM, D, blk):
  grid = (B // blk,)
  row_spec = lambda w: pl.BlockSpec((blk, w), lambda i: (i, _i32(0)))
  fix_spec = lambda a, b: pl.BlockSpec((a, b), lambda i: (_i32(0), _i32(0)))
  return pl.pallas_call(
      _gru_body,
      grid=grid,
      in_specs=[
          row_spec(DM),
          row_spec(D),
          fix_spec(DM, D), fix_spec(DM, D), fix_spec(DM, D),
          fix_spec(D, D), fix_spec(D, D), fix_spec(D, D),
          fix_spec(1, D), fix_spec(1, D), fix_spec(1, D), fix_spec(1, D),
      ],
      out_specs=row_spec(2 * D),
      out_shape=jax.ShapeDtypeStruct((B, 2 * D), jnp.float32),
  )


def _scatter_body(meta_hbm, ts_hbm, newh_hbm, memt_hbm, lu_ref,
                  outt_hbm,
                  ts_v, l_b, l_id, wb, wl, win, win2, wint, rows16, ts16,
                  cbuf, sem, insem):
  D = memt_hbm.shape[0]
  M = memt_hbm.shape[1]
  nwin = (M + WC - 1) // WC
  wid = lax.axis_index("c") * NS + lax.axis_index("s")
  iota = lax.iota(jnp.int32, 16)

  pltpu.sync_copy(ts_hbm, ts_v)
  pltpu.sync_copy(meta_hbm.at[_i32(wid), pl.ds(0, LB)], l_b)
  pltpu.sync_copy(meta_hbm.at[_i32(wid), pl.ds(LB, LB)], l_id)
  pltpu.sync_copy(meta_hbm.at[_i32(wid), pl.ds(2 * LB, 128)], cbuf)
  cnt = cbuf[pl.ds(0, 16)][0]
  nscan = (cnt + 15) // 16

  def lu16(k):
    off = k * 16
    valid = (off + iota) < cnt
    bv = l_b[pl.ds(off, 16)]
    idv = l_id[pl.ds(off, 16)]
    ts16[pl.ds(0, 16)] = plsc.load_gather(ts_v, [jnp.where(valid, bv, 0)])
    pltpu.async_copy(
        ts16, lu_ref.at[plsc.Indices(jnp.where(valid, idv, -1),
                                     ignored_value=-1)], sem).wait()
  _floop(nscan, lu16)

  def process(widx, wref):
    c0 = widx * WC

    def scan(i, wcnt):
      off = i * 16
      valid = (off + iota) < cnt
      idv = l_id[pl.ds(off, 16)]
      inwin = valid & ((idv // WC) == widx)
      plsc.store_compressed(wb.at[pl.ds(wcnt, 16)], l_b[pl.ds(off, 16)],
                            mask=inwin)
      plsc.store_compressed(wl.at[pl.ds(wcnt, 16)],
                            jnp.where(inwin, idv - c0, 0), mask=inwin)
      return wcnt + plsc.all_reduce_population_count(inwin)[0]

    wcnt = lax.fori_loop(_i32(0), nscan, scan, _i32(0))

    def apply16(k):
      off = k * 16
      valid = (off + iota) < wcnt
      bv = wb[pl.ds(off, 16)]
      local = wl[pl.ds(off, 16)]
      pltpu.async_copy(
          newh_hbm.at[plsc.Indices(jnp.where(valid, bv, -1),
                                   ignored_value=-1)],
          rows16, sem).wait()
      for j in range(16):
        @pl.when(off + j < wcnt)
        def _apply(j=j):
          colv = jnp.full((16,), local[j], jnp.int32)
          for q in range(4):
            plsc.store_scatter(wref, [iota + q * 16, colv],
                               rows16[j, pl.ds(q * 16, 16)])

    _floop((wcnt + 15) // 16, apply16)

  gmax = (_i32(nwin - 1) - wid) // NW + 1

  def in_cp(g, buf):
    widx = wid + g * NW
    c0 = widx * WC
    return pltpu.make_async_copy(memt_hbm.at[:, pl.ds(c0, WC)], buf, insem)

  def tail_cp(g):
    c0 = (wid + g * NW) * WC
    return pltpu.make_async_copy(memt_hbm.at[:, pl.ds(c0, 128)], wint, insem)

  def start_next(g, buf):
    @pl.when(g < gmax)
    def _():
      is_tail = wid + g * NW == nwin - 1

      @pl.when(jnp.logical_not(is_tail))
      def _f():
        in_cp(g, buf).start()

      @pl.when(is_tail)
      def _t():
        tail_cp(g).start()

  def run_window(g, buf):
    @pl.when(g < gmax)
    def _():
      widx = wid + g * NW
      c0 = widx * WC
      is_tail = widx == nwin - 1

      @pl.when(jnp.logical_not(is_tail))
      def _f():
        in_cp(g, buf).wait()
        process(widx, buf)
        pltpu.sync_copy(buf, outt_hbm.at[:, pl.ds(c0, WC)])

      @pl.when(is_tail)
      def _t():
        tail_cp(g).wait()
        process(widx, wint)
        pltpu.sync_copy(wint, outt_hbm.at[:, pl.ds(c0, 128)])

  start_next(_i32(0), win)

  def pair(t):
    g0 = t * 2
    start_next(g0 + 1, win2)
    run_window(g0, win)
    start_next(g0 + 2, win)
    run_window(g0 + 1, win2)

  _floop((gmax + 1) // 2, pair)


def _make_scatter(M, D, DM, B):
  mesh = plsc.VectorSubcoreMesh(core_axis_name="c", subcore_axis_name="s")
  return pl.kernel(
      _scatter_body,
      out_type=jax.ShapeDtypeStruct((D, M), jnp.float32),
      mesh=mesh,
      compiler_params=pltpu.CompilerParams(needs_layout_passes=False,
                                           disable_bounds_checks=True),
      scratch_types=[
          pltpu.VMEM((B,), jnp.float32),
          pltpu.VMEM((LB,), jnp.int32),
          pltpu.VMEM((LB,), jnp.int32),
          pltpu.VMEM((WC + 16,), jnp.int32),
          pltpu.VMEM((WC + 16,), jnp.int32),
          pltpu.VMEM((D, WC), jnp.float32),
          pltpu.VMEM((D, WC), jnp.float32),
          pltpu.VMEM((D, 128), jnp.float32),
          pltpu.VMEM((16, DM), jnp.float32),
          pltpu.VMEM((16,), jnp.float32),
          pltpu.VMEM((128,), jnp.int32),
          pltpu.SemaphoreType.DMA,
          pltpu.SemaphoreType.DMA,
      ],
  )


def kernel(unique_node_ids, unique_messages, timestamps, memory, last_update,
           W_ih, W_hh, b_ih, b_hh):
  M, D = memory.shape
  B, DM = unique_messages.shape

  ids2 = unique_node_ids.astype(jnp.int32).reshape(B // 128, 128)
  W_ihT = W_ih.T.astype(jnp.float32)
  W_hhT = W_hh.T.astype(jnp.float32)
  wir, wiz, win = W_ihT[:, :D], W_ihT[:, D:2 * D], W_ihT[:, 2 * D:]
  whr, whz, whn = W_hhT[:, :D], W_hhT[:, D:2 * D], W_hhT[:, 2 * D:]
  br = (b_ih[:D] + b_hh[:D]).reshape(1, D)
  bz = (b_ih[D:2 * D] + b_hh[D:2 * D]).reshape(1, D)
  bni = b_ih[2 * D:].reshape(1, D)
  bnh = b_hh[2 * D:].reshape(1, D)

  mem_t = memory.T

  h, meta = _make_gather(M, D, B)(ids2, mem_t)
  newh = _make_gru(B, DM, D, 512)(
      unique_messages, h, wir, wiz, win, whr, whz, whn, br, bz, bni, bnh)

  lu_ref = jax.new_ref(last_update)
  out_t = _make_scatter(M, D, DM, B)(meta, timestamps, newh, mem_t, lu_ref)
  return out_t.T, lu_ref[...]


# speedup vs baseline: 1.0509x; 1.0031x over previous
import functools

import jax
import jax.numpy as jnp
from jax import lax
from jax.experimental import pallas as pl
from jax.experimental.pallas import tpu as pltpu
from jax.experimental.pallas import tpu_sc as plsc

NC = 2
NS = 16
NW = NC * NS
WC = 512


def _i32(x):
  return jnp.asarray(x, jnp.int32)


def _floop(n, body):
  lax.fori_loop(_i32(0), _i32(n), lambda i, c: (body(i), c)[1], _i32(0),
                unroll=False)


LB = 16512
MW = 2 * LB + 128


def _win_owner(idv, wid):
  winidx = idv // WC
  return winidx, lax.rem(winidx, _i32(NW)) == wid


def _gather_body(ids2_hbm, memt_hbm, h_hbm, meta_hbm,
                 l_b, l_id, wb, wl, win, win2, wint, hst, cbuf, sem, insem):
  rows = ids2_hbm.shape[0]
  M = memt_hbm.shape[1]
  nwin = (M + WC - 1) // WC
  TBL = ((nwin + NW - 1) // NW) * WC
  wid = lax.axis_index("c") * NS + lax.axis_index("s")
  iota = lax.iota(jnp.int32, 16)
  zeros16 = jnp.zeros((16,), jnp.int32)

  def passes(ids_v, tbl):
    def zero(i):
      tbl[pl.ds(i * 16, 16)] = zeros16
    _floop(TBL // 16, zero)

    pltpu.sync_copy(ids2_hbm, ids_v)

    def slot_of(idv):
      winidx, m = _win_owner(idv, wid)
      slot = (winidx // NW) * WC + lax.rem(idv, _i32(WC))
      return m, jnp.where(m, slot, 0)

    def p1(r):
      for o in range(8):
        b0 = r * 128 + o * 16
        idv = ids_v[r, pl.ds(o * 16, 16)]
        m, slot = slot_of(idv)
        plsc.store_scatter(tbl, [slot], iota + (b0 + 1), mask=m)
    _floop(rows, p1)

    def p2(r, cnt):
      for o in range(8):
        b0 = r * 128 + o * 16
        idv = ids_v[r, pl.ds(o * 16, 16)]
        m, slot = slot_of(idv)
        g = plsc.load_gather(tbl, [slot], mask=m)
        keep = m & (g == iota + (b0 + 1))
        plsc.store_compressed(l_b.at[pl.ds(cnt, 16)], iota + b0, mask=keep)
        plsc.store_compressed(l_id.at[pl.ds(cnt, 16)], idv, mask=keep)
        cnt = cnt + plsc.all_reduce_population_count(keep)[0]
      return cnt

    return lax.fori_loop(_i32(0), _i32(rows), p2, _i32(0))

  cnt = pl.run_scoped(
      passes,
      pltpu.VMEM((rows, 128), jnp.int32),
      pltpu.VMEM((TBL,), jnp.int32))
  nscan = (cnt + 15) // 16

  cbuf[pl.ds(0, 16)] = jnp.full((16,), cnt, jnp.int32)
  pltpu.sync_copy(l_b, meta_hbm.at[_i32(wid), pl.ds(0, LB)])
  pltpu.sync_copy(l_id, meta_hbm.at[_i32(wid), pl.ds(LB, LB)])
  pltpu.sync_copy(cbuf, meta_hbm.at[_i32(wid), pl.ds(2 * LB, 128)])

  def extract(sub, wref):
    c0 = sub * (WC // 2)

    def scan(i, wcnt):
      off = i * 16
      valid = (off + iota) < cnt
      idv = l_id[pl.ds(off, 16)]
      inwin = valid & ((idv // (WC // 2)) == sub)
      plsc.store_compressed(wb.at[pl.ds(wcnt, 16)], l_b[pl.ds(off, 16)],
                            mask=inwin)
      plsc.store_compressed(wl.at[pl.ds(wcnt, 16)],
                            jnp.where(inwin, idv - c0, 0), mask=inwin)
      return wcnt + plsc.all_reduce_population_count(inwin)[0]

    wcnt = lax.fori_loop(_i32(0), nscan, scan, _i32(0))

    def ext16(k):
      off = k * 16
      bv = wb[pl.ds(off, 16)]
      lv = wl[pl.ds(off, 16)]
      cps = []
      for j in range(16):
        ok = off + j < wcnt

        @pl.when(ok)
        def _pull(j=j):
          colv = jnp.full((16,), lv[j], jnp.int32)
          for q in range(4):
            hst[j, pl.ds(q * 16, 16)] = plsc.load_gather(
                wref, [iota + q * 16, colv])
        cp = pltpu.make_async_copy(
            hst.at[pl.ds(j, 1)], h_hbm.at[pl.ds(bv[j], 1)], sem)
        cps.append((ok, cp))
      for ok, cp in cps:
        @pl.when(ok)
        def _start(cp=cp):
          cp.start()
      for ok, cp in cps:
        @pl.when(ok)
        def _wait(cp=cp):
          cp.wait()

    _floop((wcnt + 15) // 16, ext16)

  gmax = (_i32(nwin - 1) - wid) // NW + 1
  owns_tail = (wid == lax.rem(_i32(nwin - 1), _i32(NW))).astype(jnp.int32)
  hmax = 2 * gmax - owns_tail

  def sub_of(h):
    return 2 * wid + (h // 2) * (2 * NW) + lax.rem(h, _i32(2))

  def in_cp(h, buf):
    c0 = sub_of(h) * (WC // 2)
    return pltpu.make_async_copy(
        memt_hbm.at[:, pl.ds(c0, WC // 2)], buf, insem)

  def tail_cp(h):
    c0 = sub_of(h) * (WC // 2)
    return pltpu.make_async_copy(memt_hbm.at[:, pl.ds(c0, 128)], wint, insem)

  def is_tail_sub(h):
    return (owns_tail == 1) & (h == hmax - 1)

  def start_next(h, buf):
    @pl.when(h < hmax)
    def _():
      @pl.when(jnp.logical_not(is_tail_sub(h)))
      def _f():
        in_cp(h, buf).start()

      @pl.when(is_tail_sub(h))
      def _t():
        tail_cp(h).start()

  def run_window(h, buf):
    @pl.when(h < hmax)
    def _():
      @pl.when(jnp.logical_not(is_tail_sub(h)))
      def _f():
        in_cp(h, buf).wait()
        extract(sub_of(h), buf)

      @pl.when(is_tail_sub(h))
      def _t():
        tail_cp(h).wait()
        extract(sub_of(h), wint)

  start_next(_i32(0), win)

  def pair(t):
    h0 = t * 2
    start_next(h0 + 1, win2)
    run_window(h0, win)
    start_next(h0 + 2, win)
    run_window(h0 + 1, win2)

  _floop((hmax + 1) // 2, pair)


def _make_gather(M, D, B):
  rows = B // 128
  nwin = (M + WC - 1) // WC
  TBL = ((nwin + NW - 1) // NW) * WC
  mesh = plsc.VectorSubcoreMesh(core_axis_name="c", subcore_axis_name="s")
  return pl.kernel(
      _gather_body,
      out_type=(jax.ShapeDtypeStruct((B, D), jnp.float32),
                jax.ShapeDtypeStruct((NW, MW), jnp.int32)),
      mesh=mesh,
      compiler_params=pltpu.CompilerParams(needs_layout_passes=False,
                                           disable_bounds_checks=True),
      scratch_types=[
          pltpu.VMEM((LB,), jnp.int32),
          pltpu.VMEM((LB,), jnp.int32),
          pltpu.VMEM((WC + 16,), jnp.int32),
          pltpu.VMEM((WC + 16,), jnp.int32),
          pltpu.VMEM((D, WC // 2), jnp.float32),
          pltpu.VMEM((D, WC // 2), jnp.float32),
          pltpu.VMEM((D, 128), jnp.float32),
          pltpu.VMEM((16, D), jnp.float32),
          pltpu.VMEM((128,), jnp.int32),
          pltpu.SemaphoreType.DMA,
          pltpu.SemaphoreType.DMA,
      ],
  )


def _gru_body(x_ref, h_ref, wir, wiz, win, whr, whz, whn, br, bz, bni, bnh,
              out_ref):
  x = x_ref[...]
  h = h_ref[...]
  dot = functools.partial(
      lax.dot_general,
      dimension_numbers=(((1,), (0,)), ((), ())),
      preferred_element_type=jnp.float32)
  r = jax.nn.sigmoid(dot(x, wir[...]) + dot(h, whr[...]) + br[...])
  z = jax.nn.sigmoid(dot(x, wiz[...]) + dot(h, whz[...]) + bz[...])
  n = jnp.tanh(dot(x, win[...]) + bni[...] + r * (dot(h, whn[...]) + bnh[...]))
  res = (1.0 - z) * n + z * h
  out_ref[...] = jnp.concatenate([res, jnp.zeros_like(res)], axis=1)


def _make_gru(B, DM, D, blk):
  grid = (B // blk,)
  row_spec = lambda w: pl.BlockSpec((blk, w), lambda i: (i, _i32(0)))
  fix_spec = lambda a, b: pl.BlockSpec((a, b), lambda i: (_i32(0), _i32(0)))
  return pl.pallas_call(
      _gru_body,
      grid=grid,
      in_specs=[
          row_spec(DM),
          row_spec(D),
          fix_spec(DM, D), fix_spec(DM, D), fix_spec(DM, D),
          fix_spec(D, D), fix_spec(D, D), fix_spec(D, D),
          fix_spec(1, D), fix_spec(1, D), fix_spec(1, D), fix_spec(1, D),
      ],
      out_specs=row_spec(2 * D),
      out_shape=jax.ShapeDtypeStruct((B, 2 * D), jnp.float32),
  )


def _scatter_body(meta_hbm, ts_hbm, newh_hbm, memt_hbm, lu_ref,
                  outt_hbm,
                  ts_v, l_b, l_id, wb, wl, win, win2, wint, rows16, ts16,
                  cbuf, sem, insem):
  D = memt_hbm.shape[0]
  M = memt_hbm.shape[1]
  nwin = (M + WC - 1) // WC
  wid = lax.axis_index("c") * NS + lax.axis_index("s")
  iota = lax.iota(jnp.int32, 16)

  pltpu.sync_copy(ts_hbm, ts_v)
  pltpu.sync_copy(meta_hbm.at[_i32(wid), pl.ds(0, LB)], l_b)
  pltpu.sync_copy(meta_hbm.at[_i32(wid), pl.ds(LB, LB)], l_id)
  pltpu.sync_copy(meta_hbm.at[_i32(wid), pl.ds(2 * LB, 128)], cbuf)
  cnt = cbuf[pl.ds(0, 16)][0]
  nscan = (cnt + 15) // 16

  def lu128(t):
    cps = []
    for u in range(8):
      k = t * 8 + u
      off = k * 16
      ok = k < nscan

      @pl.when(ok)
      def _build(u=u, off=off):
        valid = (off + iota) < cnt
        bv = l_b[pl.ds(off, 16)]
        ts16[u, pl.ds(0, 16)] = plsc.load_gather(
            ts_v, [jnp.where(valid, bv, 0)])

      valid = (off + iota) < cnt
      idv = l_id[pl.ds(off, 16)]
      cp = pltpu.make_async_copy(
          ts16.at[_i32(u)],
          lu_ref.at[plsc.Indices(jnp.where(valid, idv, -1),
                                 ignored_value=-1)], sem)
      cps.append((ok, cp))
    for ok, cp in cps:
      @pl.when(ok)
      def _start(cp=cp):
        cp.start()
    for ok, cp in cps:
      @pl.when(ok)
      def _wait(cp=cp):
        cp.wait()
  _floop((nscan + 7) // 8, lu128)

  def process(widx, wref):
    c0 = widx * WC

    def scan(i, wcnt):
      off = i * 16
      valid = (off + iota) < cnt
      idv = l_id[pl.ds(off, 16)]
      inwin = valid & ((idv // WC) == widx)
      plsc.store_compressed(wb.at[pl.ds(wcnt, 16)], l_b[pl.ds(off, 16)],
                            mask=inwin)
      plsc.store_compressed(wl.at[pl.ds(wcnt, 16)],
                            jnp.where(inwin, idv - c0, 0), mask=inwin)
      return wcnt + plsc.all_reduce_population_count(inwin)[0]

    wcnt = lax.fori_loop(_i32(0), nscan, scan, _i32(0))

    def apply16(k):
      off = k * 16
      valid = (off + iota) < wcnt
      bv = wb[pl.ds(off, 16)]
      local = wl[pl.ds(off, 16)]
      pltpu.async_copy(
          newh_hbm.at[plsc.Indices(jnp.where(valid, bv, -1),
                                   ignored_value=-1)],
          rows16, sem).wait()
      for j in range(16):
        @pl.when(off + j < wcnt)
        def _apply(j=j):
          colv = jnp.full((16,), local[j], jnp.int32)
          for q in range(4):
            plsc.store_scatter(wref, [iota + q * 16, colv],
                               rows16[j, pl.ds(q * 16, 16)])

    _floop((wcnt + 15) // 16, apply16)

  gmax = (_i32(nwin - 1) - wid) // NW + 1

  def in_cp(g, buf):
    widx = wid + g * NW
    c0 = widx * WC
    return pltpu.make_async_copy(memt_hbm.at[:, pl.ds(c0, WC)], buf, insem)

  def tail_cp(g):
    c0 = (wid + g * NW) * WC
    return pltpu.make_async_copy(memt_hbm.at[:, pl.ds(c0, 128)], wint, insem)

  def start_next(g, buf):
    @pl.when(g < gmax)
    def _():
      is_tail = wid + g * NW == nwin - 1

      @pl.when(jnp.logical_not(is_tail))
      def _f():
        in_cp(g, buf).start()

      @pl.when(is_tail)
      def _t():
        tail_cp(g).start()

  def run_window(g, buf):
    @pl.when(g < gmax)
    def _():
      widx = wid + g * NW
      c0 = widx * WC
      is_tail = widx == nwin - 1

      @pl.when(jnp.logical_not(is_tail))
      def _f():
        in_cp(g, buf).wait()
        process(widx, buf)
        pltpu.sync_copy(buf, outt_hbm.at[:, pl.ds(c0, WC)])

      @pl.when(is_tail)
      def _t():
        tail_cp(g).wait()
        process(widx, wint)
        pltpu.sync_copy(wint, outt_hbm.at[:, pl.ds(c0, 128)])

  start_next(_i32(0), win)

  def pair(t):
    g0 = t * 2
    start_next(g0 + 1, win2)
    run_window(g0, win)
    start_next(g0 + 2, win)
    run_window(g0 + 1, win2)

  _floop((gmax + 1) // 2, pair)


def _make_scatter(M, D, DM, B):
  mesh = plsc.VectorSubcoreMesh(core_axis_name="c", subcore_axis_name="s")
  return pl.kernel(
      _scatter_body,
      out_type=jax.ShapeDtypeStruct((D, M), jnp.float32),
      mesh=mesh,
      compiler_params=pltpu.CompilerParams(needs_layout_passes=False,
                                           disable_bounds_checks=True),
      scratch_types=[
          pltpu.VMEM((B,), jnp.float32),
          pltpu.VMEM((LB,), jnp.int32),
          pltpu.VMEM((LB,), jnp.int32),
          pltpu.VMEM((WC + 16,), jnp.int32),
          pltpu.VMEM((WC + 16,), jnp.int32),
          pltpu.VMEM((D, WC), jnp.float32),
          pltpu.VMEM((D, WC), jnp.float32),
          pltpu.VMEM((D, 128), jnp.float32),
          pltpu.VMEM((16, DM), jnp.float32),
          pltpu.VMEM((8, 16), jnp.float32),
          pltpu.VMEM((128,), jnp.int32),
          pltpu.SemaphoreType.DMA,
          pltpu.SemaphoreType.DMA,
      ],
  )


def kernel(unique_node_ids, unique_messages, timestamps, memory, last_update,
           W_ih, W_hh, b_ih, b_hh):
  M, D = memory.shape
  B, DM = unique_messages.shape

  ids2 = unique_node_ids.astype(jnp.int32).reshape(B // 128, 128)
  W_ihT = W_ih.T.astype(jnp.float32)
  W_hhT = W_hh.T.astype(jnp.float32)
  wir, wiz, win = W_ihT[:, :D], W_ihT[:, D:2 * D], W_ihT[:, 2 * D:]
  whr, whz, whn = W_hhT[:, :D], W_hhT[:, D:2 * D], W_hhT[:, 2 * D:]
  br = (b_ih[:D] + b_hh[:D]).reshape(1, D)
  bz = (b_ih[D:2 * D] + b_hh[D:2 * D]).reshape(1, D)
  bni = b_ih[2 * D:].reshape(1, D)
  bnh = b_hh[2 * D:].reshape(1, D)

  mem_t = memory.T

  h, meta = _make_gather(M, D, B)(ids2, mem_t)
  newh = _make_gru(B, DM, D, 512)(
      unique_messages, h, wir, wiz, win, whr, whz, whn, br, bz, bni, bnh)

  lu_ref = jax.new_ref(last_update)
  out_t = _make_scatter(M, D, DM, B)(meta, timestamps, newh, mem_t, lu_ref)
  return out_t.T, lu_ref[...]
